# baseline (speedup 1.0000x reference)
import functools

import jax
import jax.numpy as jnp
from jax import lax
from jax.experimental import pallas as pl
from jax.experimental.pallas import tpu as pltpu
from jax.experimental.pallas import tpu_sc as plsc

DIM = 64
PDIM = 128
VROWS = 2_000_000
NUM_CORES = 2
NUM_SUBCORES = 16
NW = NUM_CORES * NUM_SUBCORES
TOTAL = 4096 * 200
PER_W = TOTAL // NW
CHUNK = 320
NCHUNK = PER_W // CHUNK
NBUF = 4
NGRP = NCHUNK // NBUF

_mesh = plsc.VectorSubcoreMesh(core_axis_name="c", subcore_axis_name="s")


@functools.partial(
    pl.kernel,
    out_type=jax.ShapeDtypeStruct((TOTAL, PDIM), jnp.float32),
    mesh=_mesh,
    scratch_types=[
        pltpu.VMEM((PER_W,), jnp.int32),
        pltpu.VMEM((NBUF, CHUNK, DIM), jnp.float32),
        pltpu.SemaphoreType.DMA((NBUF,)),
        pltpu.SemaphoreType.DMA((NBUF,)),
    ],
    compiler_params=pltpu.CompilerParams(use_tc_tiling_on_sc=False),
)
def _gather_kernel(idx_hbm, table_hbm, out_hbm, idx_v, rows_v, gsem, ssem):
    wid = lax.axis_index("s") * NUM_CORES + lax.axis_index("c")
    base = wid * PER_W

    pltpu.sync_copy(idx_hbm.at[pl.ds(base, PER_W)], idx_v)

    def gather(j, b):
        pltpu.async_copy(
            table_hbm.at[idx_v.at[pl.ds(j * CHUNK, CHUNK)]], rows_v.at[b],
            gsem.at[b])

    def store(j, b):
        pltpu.async_copy(
            rows_v.at[b],
            out_hbm.at[pl.ds(base + j * CHUNK, CHUNK), pl.ds(0, DIM)],
            ssem.at[b])

    def wait_g(b):
        pltpu.make_async_copy(
            table_hbm.at[idx_v.at[pl.ds(0, CHUNK)]], rows_v.at[b],
            gsem.at[b]).wait()

    def wait_s(b):
        pltpu.make_async_copy(
            rows_v.at[b],
            out_hbm.at[pl.ds(base, CHUNK), pl.ds(0, DIM)],
            ssem.at[b]).wait()

    for b in range(NBUF):
        gather(b, b)

    def body(g, carry):
        j0 = NBUF * g
        for b in range(NBUF):
            wait_g(b)
            store(j0 + b, b)

        @pl.when(g < NGRP - 1)
        def _():
            for b in range(NBUF):
                wait_s(b)
                gather(j0 + NBUF + b, b)

        return carry

    lax.fori_loop(0, NGRP, body, 0)
    for b in range(NBUF):
        wait_s(b)


def kernel(input_ids, table):
    flat2 = (input_ids.reshape(-1) * 2).astype(jnp.int32)
    table_p = jnp.pad(table, ((0, 0), (0, PDIM - DIM)))
    table_v = table_p.reshape(VROWS, DIM)
    out = _gather_kernel(flat2, table_v)
    return out[:, :DIM].reshape(input_ids.shape + (DIM,))


# speedup vs baseline: 1.0029x; 1.0029x over previous
import functools

import jax
import jax.numpy as jnp
from jax import lax
from jax.experimental import pallas as pl
from jax.experimental.pallas import tpu as pltpu
from jax.experimental.pallas import tpu_sc as plsc

DIM = 64
PDIM = 128
VROWS = 2_000_000
NUM_CORES = 2
NUM_SUBCORES = 16
NW = NUM_CORES * NUM_SUBCORES
TOTAL = 4096 * 200
PER_W = TOTAL // NW
CHUNK = 320
NCHUNK = PER_W // CHUNK
NBUF = 5
NGRP = NCHUNK // NBUF

_mesh = plsc.VectorSubcoreMesh(core_axis_name="c", subcore_axis_name="s")


@functools.partial(
    pl.kernel,
    out_type=jax.ShapeDtypeStruct((TOTAL, PDIM), jnp.float32),
    mesh=_mesh,
    scratch_types=[
        pltpu.VMEM((PER_W,), jnp.int32),
        pltpu.VMEM((NBUF, CHUNK, DIM), jnp.float32),
        pltpu.SemaphoreType.DMA((NBUF,)),
        pltpu.SemaphoreType.DMA((NBUF,)),
    ],
    compiler_params=pltpu.CompilerParams(use_tc_tiling_on_sc=False),
)
def _gather_kernel(idx_hbm, table_hbm, out_hbm, idx_v, rows_v, gsem, ssem):
    wid = lax.axis_index("s") * NUM_CORES + lax.axis_index("c")
    base = wid * PER_W

    pltpu.sync_copy(idx_hbm.at[pl.ds(base, PER_W)], idx_v)

    def gather(j, b):
        pltpu.async_copy(
            table_hbm.at[idx_v.at[pl.ds(j * CHUNK, CHUNK)]], rows_v.at[b],
            gsem.at[b])

    def store(j, b):
        pltpu.async_copy(
            rows_v.at[b],
            out_hbm.at[pl.ds(base + j * CHUNK, CHUNK), pl.ds(0, DIM)],
            ssem.at[b])

    def wait_g(b):
        pltpu.make_async_copy(
            table_hbm.at[idx_v.at[pl.ds(0, CHUNK)]], rows_v.at[b],
            gsem.at[b]).wait()

    def wait_s(b):
        pltpu.make_async_copy(
            rows_v.at[b],
            out_hbm.at[pl.ds(base, CHUNK), pl.ds(0, DIM)],
            ssem.at[b]).wait()

    for b in range(NBUF):
        gather(b, b)

    def body(g, carry):
        j0 = NBUF * g
        for b in range(NBUF):
            wait_g(b)
            store(j0 + b, b)

        @pl.when(g < NGRP - 1)
        def _():
            for b in range(NBUF):
                wait_s(b)
                gather(j0 + NBUF + b, b)

        return carry

    lax.fori_loop(0, NGRP, body, 0)
    for b in range(NBUF):
        wait_s(b)


def kernel(input_ids, table):
    flat2 = (input_ids.reshape(-1) * 2).astype(jnp.int32)
    table_p = jnp.pad(table, ((0, 0), (0, PDIM - DIM)))
    table_v = table_p.reshape(VROWS, DIM)
    out = _gather_kernel(flat2, table_v)
    return out[:, :DIM].reshape(input_ids.shape + (DIM,))
